# Initial kernel scaffold; baseline (speedup 1.0000x reference)
import jax
import jax.numpy as jnp
from jax.experimental import pallas as pl


def kernel(indices, table):
    raise NotImplementedError("write your pallas kernel here")


# speedup vs baseline: 1.6709x; 1.6709x over previous
import functools

import jax
import jax.numpy as jnp
from jax import lax
from jax.experimental import pallas as pl
from jax.experimental.pallas import tpu as pltpu
from jax.experimental.pallas import tpu_sc as plsc

VOCAB = 21
EMB_DIM = 128
BATCH = 4096
SEQ_LEN = 50

_info = plsc.get_sparse_core_info()
_NC = _info.num_cores
_NS = _info.num_subcores
_NW = _NC * _NS
_BCHUNK = BATCH // _NW
_LANES = 16


def _emb_body(idx_hbm, table_hbm, out_hbm, patt_v, idxcol_v, rows_v,
              sem_i, sem_r):
    wid = lax.axis_index("s") * _NC + lax.axis_index("c")
    b0 = wid * _BCHUNK
    lane = lax.iota(jnp.int32, _LANES)

    def body(l, carry):
        for k in range(_BCHUNK // _LANES):
            patt_v[pl.ds(k * _LANES, _LANES)] = (
                (b0 + k * _LANES + lane) * SEQ_LEN + l)
        pltpu.async_copy(idx_hbm.at[patt_v], idxcol_v, sem_i).wait()
        pltpu.async_copy(table_hbm.at[idxcol_v], rows_v, sem_r).wait()
        pltpu.sync_copy(rows_v, out_hbm.at[l, pl.ds(b0, _BCHUNK), :])
        return carry

    lax.fori_loop(0, SEQ_LEN, body, 0)


_emb_kernel = functools.partial(
    pl.kernel,
    mesh=plsc.VectorSubcoreMesh(core_axis_name="c", subcore_axis_name="s"),
    out_type=jax.ShapeDtypeStruct((SEQ_LEN, BATCH, EMB_DIM), jnp.float32),
    scratch_types=[
        pltpu.VMEM((_BCHUNK,), jnp.int32),
        pltpu.VMEM((_BCHUNK,), jnp.int32),
        pltpu.VMEM((_BCHUNK, EMB_DIM), jnp.float32),
        pltpu.SemaphoreType.DMA,
        pltpu.SemaphoreType.DMA,
    ],
)(_emb_body)


def kernel(indices, table):
    return _emb_kernel(indices.astype(jnp.int32).reshape(-1), table)
